# Initial kernel scaffold; baseline (speedup 1.0000x reference)
import jax
import jax.numpy as jnp
from jax.experimental import pallas as pl


def kernel(in_feat, edge_index, W1, b1, W2, b2, W3, b3, W4, b4):
    raise NotImplementedError("write your pallas kernel here")


# speedup vs baseline: 2.7127x; 2.7127x over previous
import functools

import jax
import jax.numpy as jnp
from jax import lax
from jax.experimental import pallas as pl
from jax.experimental.pallas import tpu as pltpu
from jax.experimental.pallas import tpu_sc as plsc

F = 128
CHUNK = 128
NW = 32

_TH = ((3.0, -3.0, 0.75), (0.0, 3.0, -1.5), (0.0, 0.0, 0.75))


def _round_up(x, m):
    return ((x + m - 1) // m) * m


@functools.lru_cache(maxsize=None)
def _make_prop(ep, npad):
    epw = ep // NW
    nchunks = epw // CHUNK
    rpt = npad // 16
    nzfull, nzrem = divmod(rpt, CHUNK)
    mesh = plsc.VectorSubcoreMesh(core_axis_name="c", subcore_axis_name="s")

    def body(g_hbm, src_hbm, dst_hbm, out0_hbm, out1_hbm,
             sv, dv, rows, zbuf, acc, sem):
        cid = lax.axis_index("c")
        sid = lax.axis_index("s")
        zero16 = jnp.zeros((16,), jnp.float32)

        def fill_z(i, c):
            for j in range(F // 16):
                zbuf[i, pl.ds(j * 16, 16)] = zero16
            return c
        lax.fori_loop(0, CHUNK, fill_z, 0)

        r0 = sid * rpt
        for k in range(nzfull):
            pltpu.sync_copy(zbuf, acc.at[pl.ds(r0 + k * CHUNK, CHUNK)])
        if nzrem:
            pltpu.sync_copy(zbuf.at[pl.ds(0, nzrem)],
                            acc.at[pl.ds(r0 + nzfull * CHUNK, nzrem)])
        plsc.subcore_barrier()

        base = (cid * 16 + sid) * epw

        def step(i, c):
            off = base + i * CHUNK
            pltpu.sync_copy(src_hbm.at[pl.ds(off, CHUNK)], sv)
            pltpu.async_copy(g_hbm.at[sv], rows, sem).wait()
            pltpu.sync_copy(dst_hbm.at[pl.ds(off, CHUNK)], dv)
            pltpu.sync_copy(rows, acc.at[dv], add=True)
            return c
        lax.fori_loop(0, nchunks, step, 0)

        plsc.subcore_barrier()

        @pl.when(cid == 0)
        def _():
            pltpu.sync_copy(acc.at[pl.ds(r0, rpt)], out0_hbm.at[pl.ds(r0, rpt)])

        @pl.when(cid == 1)
        def _():
            pltpu.sync_copy(acc.at[pl.ds(r0, rpt)], out1_hbm.at[pl.ds(r0, rpt)])

    return pl.kernel(
        body,
        out_type=[jax.ShapeDtypeStruct((npad, F), jnp.float32),
                  jax.ShapeDtypeStruct((npad, F), jnp.float32)],
        mesh=mesh,
        scratch_types=[
            pltpu.VMEM((CHUNK,), jnp.int32),
            pltpu.VMEM((CHUNK,), jnp.int32),
            pltpu.VMEM((CHUNK, F), jnp.float32),
            pltpu.VMEM((CHUNK, F), jnp.float32),
            pltpu.VMEM_SHARED((npad, F), jnp.float32),
            pltpu.SemaphoreType.DMA,
        ],
    )


def _mm_t(x, w):
    return lax.dot_general(x, w, (((1,), (1,)), ((), ())),
                           preferred_element_type=jnp.float32)


def _pre_body(nrows, bn, x_ref, w1_ref, b1_ref, w2_ref, b2_ref,
              dp0_ref, dp1_ref, h_ref, g_ref, di_ref):
    i = pl.program_id(0)
    x = x_ref[...]
    h1 = jax.nn.relu(_mm_t(x, w1_ref[...]) + b1_ref[...])
    h2 = jax.nn.relu(_mm_t(h1, w2_ref[...]) + b2_ref[...])
    deg = dp0_ref[...][:, 0:1] + dp1_ref[...][:, 0:1]
    dinv = lax.rsqrt(jnp.maximum(deg, 1.0))
    rows = lax.broadcasted_iota(jnp.int32, (bn, 1), 0) + i * bn
    mask = (rows < nrows).astype(jnp.float32)
    h2 = h2 * mask
    h_ref[...] = h2
    g_ref[...] = h2 * dinv
    di_ref[...] = jnp.broadcast_to(dinv, (bn, 16))


def _mid_body(nrows, bn, h_ref, p0_ref, p1_ref, di_ref, c_ref, g_ref):
    i = pl.program_id(0)
    dinv = di_ref[...][:, 0:1]
    cur = h_ref[...] - (p0_ref[...] + p1_ref[...]) * dinv
    rows = lax.broadcasted_iota(jnp.int32, (bn, 1), 0) + i * bn
    mask = (rows < nrows).astype(jnp.float32)
    cur = cur * mask
    c_ref[...] = cur
    g_ref[...] = cur * dinv


def _post_body(h_ref, c1_ref, p0_ref, p1_ref, di_ref, w3_ref, b3_ref,
               w4a_ref, w4b_ref, b4_ref, hl_ref, hh_ref):
    dinv = di_ref[...][:, 0:1]
    h = h_ref[...]
    c1 = c1_ref[...]
    c2 = c1 - (p0_ref[...] + p1_ref[...]) * dinv
    o0 = _TH[0][0] * h + _TH[0][1] * c1 + _TH[0][2] * c2
    o1 = _TH[1][1] * c1 + _TH[1][2] * c2
    o2 = _TH[2][2] * c2
    hl_ref[...] = jax.nn.relu(_mm_t(o0, w3_ref[...]) + b3_ref[...])
    hh_ref[...] = jax.nn.relu(_mm_t(o1, w4a_ref[...]) + _mm_t(o2, w4b_ref[...])
                              + b4_ref[...])


def kernel(in_feat, edge_index, W1, b1, W2, b2, W3, b3, W4, b4):
    n, f = in_feat.shape
    e = edge_index.shape[1]
    npad = _round_up(n + 16, 128)
    ep = _round_up(e, NW * CHUNK)

    src = edge_index[0].astype(jnp.int32)
    dst = edge_index[1].astype(jnp.int32)
    sent = jnp.full((ep - e,), n, jnp.int32)
    srcp = jnp.concatenate([src, sent])
    dstp = jnp.concatenate([dst, sent])
    xpad = jnp.pad(in_feat, ((0, npad - n), (0, 0)))
    b1r, b2r, b3r, b4r = (x.reshape(1, f) for x in (b1, b2, b3, b4))
    W4a, W4b = W4[:, :f], W4[:, f:]
    onesg = jnp.concatenate([jnp.ones((n, f), jnp.float32),
                             jnp.zeros((npad - n, f), jnp.float32)])

    prop = _make_prop(ep, npad)
    dp0, dp1 = prop(onesg, srcp, dstp)

    bn = npad // 4
    wspec = pl.BlockSpec((f, f), lambda i: (0, 0))
    bspec = pl.BlockSpec((1, f), lambda i: (0, 0))
    rspec = pl.BlockSpec((bn, f), lambda i: (i, 0))
    dspec = pl.BlockSpec((bn, 16), lambda i: (i, 0))
    rshape = jax.ShapeDtypeStruct((npad, f), jnp.float32)

    h, g1, dinv16 = pl.pallas_call(
        functools.partial(_pre_body, n, bn),
        grid=(npad // bn,),
        in_specs=[rspec, wspec, bspec, wspec, bspec, rspec, rspec],
        out_specs=[rspec, rspec, dspec],
        out_shape=[rshape, rshape,
                   jax.ShapeDtypeStruct((npad, 16), jnp.float32)],
    )(xpad, W1, b1r, W2, b2r, dp0, dp1)

    p10, p11 = prop(g1, srcp, dstp)

    cur1, g2 = pl.pallas_call(
        functools.partial(_mid_body, n, bn),
        grid=(npad // bn,),
        in_specs=[rspec, rspec, rspec, dspec],
        out_specs=[rspec, rspec],
        out_shape=[rshape, rshape],
    )(h, p10, p11, dinv16)

    p20, p21 = prop(g2, srcp, dstp)

    hl, hh = pl.pallas_call(
        _post_body,
        grid=(npad // bn,),
        in_specs=[rspec, rspec, rspec, rspec, dspec,
                  wspec, bspec, wspec, wspec, bspec],
        out_specs=[rspec, rspec],
        out_shape=[rshape, rshape],
    )(h, cur1, p20, p21, dinv16, W3, b3r, W4a, W4b, b4r)

    return hl[:n], hh[:n]
